# baseline (speedup 1.0000x reference)
import jax
import jax.numpy as jnp
from jax import lax
from jax.experimental import pallas as pl
from jax.experimental.pallas import tpu as pltpu
from jax.experimental.pallas import tpu_sc as plsc

_B = 16384
_D = 32
_NC = 2
_NS = 16
_NW = _NC * _NS
_BPW = _B // _NW
_CHUNK = 128
_NCHUNK = _BPW // _CHUNK
_L = 16
_GPC = _CHUNK // _L
_RPB = 128 // _D


def _body(emb_hbm, blk_hbm, off_hbm, cen_hbm, out_hbm,
          idx_v, off_v, emb_v, cen_v, acc_v, sem0, sem1, sem2):
    wid = lax.axis_index("s") * _NC + lax.axis_index("c")
    base = wid * _BPW
    sems = (sem0, sem1, sem2)

    pltpu.sync_copy(blk_hbm.at[wid], idx_v)
    pltpu.sync_copy(off_hbm.at[wid], off_v)

    def fire(j):
        return pltpu.async_copy(
            cen_hbm.at[idx_v.at[j]], cen_v.at[j % 3], sems[j % 3])

    copies = [fire(0), fire(1), fire(2)]
    pltpu.sync_copy(emb_hbm.at[pl.ds(base, _BPW)], emb_v)

    lanes = lax.iota(jnp.int32, _L)
    dcols = [jnp.full((_L,), d, jnp.int32) for d in range(_D)]
    zero = jnp.zeros((_L,), jnp.float32)
    a0 = a1 = zero

    for j in range(_NCHUNK):
        copies[j].wait()
        buf = cen_v.at[j % 3]

        def group(g, accs, j=j, buf=buf):
            aa0, aa1 = accs
            rows = g * _L + lanes
            grows = (j * _CHUNK) + rows
            colbase = off_v[pl.ds(j * _CHUNK + g * _L, _L)]
            for d in range(0, _D, 2):
                e0 = plsc.load_gather(emb_v, [grows, dcols[d]])
                c0 = plsc.load_gather(buf, [rows, colbase + d])
                e1 = plsc.load_gather(emb_v, [grows, dcols[d + 1]])
                c1 = plsc.load_gather(buf, [rows, colbase + (d + 1)])
                x0 = e0 - c0
                x1 = e1 - c1
                aa0 = aa0 + x0 * x0
                aa1 = aa1 + x1 * x1
            return aa0, aa1

        a0, a1 = lax.fori_loop(0, _GPC, group, (a0, a1))
        if j + 3 < _NCHUNK:
            copies.append(fire(j + 3))

    acc_v[...] = a0 + a1
    pltpu.sync_copy(acc_v, out_hbm.at[wid])


@jax.jit
def kernel(embeddings, labels, centers):
    labels = labels.astype(jnp.int32)
    blk = (labels // _RPB).reshape(_NW, _NCHUNK, _CHUNK)
    off = ((labels % _RPB) * _D).reshape(_NW, _BPW)
    cen128 = centers.reshape(-1, 128)
    partials = pl.kernel(
        _body,
        mesh=plsc.VectorSubcoreMesh(core_axis_name="c", subcore_axis_name="s"),
        compiler_params=pltpu.CompilerParams(needs_layout_passes=False),
        out_type=jax.ShapeDtypeStruct((_NW, _L), jnp.float32),
        scratch_types=[
            pltpu.VMEM((_NCHUNK, _CHUNK), jnp.int32),
            pltpu.VMEM((_BPW,), jnp.int32),
            pltpu.VMEM((_BPW, _D), jnp.float32),
            pltpu.VMEM((3, _CHUNK, 128), jnp.float32),
            pltpu.VMEM((_L,), jnp.float32),
            pltpu.SemaphoreType.DMA,
            pltpu.SemaphoreType.DMA,
            pltpu.SemaphoreType.DMA,
        ],
    )(embeddings, blk, off, cen128)
    return jnp.sum(partials) * (1.0 / (_B * _D))


# speedup vs baseline: 3.8199x; 3.8199x over previous
import jax
import jax.numpy as jnp
from jax import lax
from jax.experimental import pallas as pl
from jax.experimental.pallas import tpu as pltpu
from jax.experimental.pallas import tpu_sc as plsc

_B = 16384
_D = 32
_V = 1000000
_NC = 2
_NS = 16
_NW = _NC * _NS
_L = 16
_WC = 256
_NWIN = 3906
_TAILC = _V - _NWIN * _WC
_NPAIRG = _NWIN // 2
_LCHUNK = 2048
_MCAP = _B + 32
_STAG = 128
_GROWS = _B + _STAG
_PTRN = 144


def _sc_body(lab_hbm, cenT_hbm, tail_hbm, g_hbm,
             labc_v, mlab_v, midx_v, csr_lab_v, csr_idx_v,
             cnt_v, ptr_v, run_v, win_v, tail_v, stag_v, sidx_v, sidx2_v,
             wsem0, wsem1, ssem):
    wid = lax.axis_index("s") * _NC + lax.axis_index("c")
    wlo = 2 * ((wid * _NPAIRG) // _NW)
    whi = 2 * (((wid + 1) * _NPAIRG) // _NW)
    nwin = whi - wlo
    lanes = lax.iota(jnp.int32, _L)
    zeros = jnp.zeros((_L,), jnp.float32)
    izeros = jnp.zeros((_L,), jnp.int32)
    dvecs = [jnp.full((_L,), d, jnp.int32) for d in range(_D)]

    pltpu.async_copy(
        cenT_hbm.at[:, pl.ds(pl.multiple_of(wlo * _WC, _WC), _WC)],
        win_v.at[0], wsem0)
    pltpu.async_copy(
        cenT_hbm.at[:, pl.ds(pl.multiple_of((wlo + 1) * _WC, _WC), _WC)],
        win_v.at[1], wsem1)
    pltpu.sync_copy(tail_hbm, tail_v)

    def zrow(r, c):
        for k in range(2, 8):
            stag_v[r, pl.ds(k * _L, _L)] = zeros
        return c
    lax.fori_loop(0, _STAG, zrow, 0)
    for k in range(8):
        sidx_v[pl.ds(k * _L, _L)] = _B + ((wid * 4 + k * _L + lanes) % _STAG)
    for k in range(_PTRN // _L):
        cnt_v[pl.ds(k * _L, _L)] = izeros

    rcal, _ = plsc.scan_count(izeros)
    rbase = rcal[0]

    iam_tail = (wid == _NW - 1)

    def scan_chunk(c, moff):
        pltpu.sync_copy(lab_hbm.at[pl.ds(c * _LCHUNK, _LCHUNK)], labc_v)

        def scan_vec(v, moff):
            l = labc_v[pl.ds(v * _L, _L)]
            gw = l >> 8
            m = (gw >= wlo) & (gw < whi)
            m = m | ((gw >= _NWIN) & iam_tail)
            ivec = c * _LCHUNK + v * _L + lanes
            plsc.store_compressed(mlab_v.at[pl.ds(moff, _L)], l, mask=m)
            plsc.store_compressed(midx_v.at[pl.ds(moff, _L)], ivec, mask=m)
            pc = plsc.all_reduce_population_count(m)
            return moff + pc[0]

        return lax.fori_loop(0, _LCHUNK // _L, scan_vec, moff)

    mcnt = lax.fori_loop(0, _B // _LCHUNK, scan_chunk, 0)
    nmv = (mcnt + _L - 1) // _L

    def hist_vec(v, c):
        valid = (v * _L + lanes) < mcnt
        l = mlab_v[pl.ds(v * _L, _L)]
        w = jnp.where(valid, jnp.minimum((l >> 8) - wlo, nwin), 0)
        rank, lastm = plsc.scan_count(w, mask=valid)
        plsc.addupdate_scatter(cnt_v, [w], rank - rbase + 1,
                               mask=lastm & valid)
        return c
    lax.fori_loop(0, nmv, hist_vec, 0)

    def prefix(k, carry):
        c = cnt_v[pl.ds(k * _L, _L)]
        s = plsc.cumsum(c)
        excl = s - c + carry
        ptr_v[pl.ds(k * _L, _L)] = excl
        run_v[pl.ds(k * _L, _L)] = excl
        return carry + s[_L - 1]
    lax.fori_loop(0, _PTRN // _L, prefix, 0)

    def csr_vec(v, c):
        valid = (v * _L + lanes) < mcnt
        l = mlab_v[pl.ds(v * _L, _L)]
        ivec = midx_v[pl.ds(v * _L, _L)]
        w = jnp.where(valid, jnp.minimum((l >> 8) - wlo, nwin), 0)
        rank, lastm = plsc.scan_count(w, mask=valid)
        base = plsc.load_gather(run_v, [w], mask=valid)
        pos = jnp.where(valid, base + rank - rbase, 0)
        plsc.store_scatter(csr_lab_v, [pos], l, mask=valid)
        plsc.store_scatter(csr_idx_v, [pos], ivec, mask=valid)
        plsc.addupdate_scatter(run_v, [w], rank - rbase + 1,
                               mask=lastm & valid)
        return c
    lax.fori_loop(0, nmv, csr_vec, 0)

    def flush():
        for k in range(8):
            sidx2_v[0, pl.ds(k * _L, _L)] = sidx_v[pl.ds(k * _L, _L)]
        pltpu.async_copy(stag_v, g_hbm.at[sidx2_v.at[0]], ssem).wait()
        for k in range(8):
            sidx_v[pl.ds(k * _L, _L)] = _B + ((wid * 4 + k * _L + lanes)
                                              % _STAG)

    def process(wl, buf, soff, tail):
        p0 = ptr_v[pl.ds(wl, _L)][0]
        cnt = cnt_v[pl.ds(wl, _L)][0]
        c0 = (wlo + wl) * _WC

        def entry_vec(v, soff):
            rem = cnt - v * _L
            valid = lanes < rem
            l = csr_lab_v[pl.ds(p0 + v * _L, _L)]
            ivec = csr_idx_v[pl.ds(p0 + v * _L, _L)]
            if tail:
                co = jnp.where(valid, l - _NWIN * _WC, 0)
            else:
                co = jnp.where(valid, l - c0, 0)
            pos = soff + plsc.cumsum(valid.astype(jnp.int32)) - 1
            pos = jnp.where(valid, pos, 0)
            for d in range(_D):
                if tail:
                    vals = plsc.load_gather(buf, [co, dvecs[d]], mask=valid)
                else:
                    vals = plsc.load_gather(buf, [dvecs[d], co], mask=valid)
                plsc.store_scatter(stag_v, [pos, dvecs[d]], vals, mask=valid)
            plsc.store_scatter(sidx_v, [pos], ivec, mask=valid)
            soff = soff + jnp.minimum(rem, _L)

            @pl.when(soff >= _STAG - _L)
            def _():
                flush()

            return jnp.where(soff >= _STAG - _L, 0, soff)

        nv = (cnt + _L - 1) // _L
        return lax.fori_loop(0, nv, entry_vec, soff)

    def pair(p, soff):
        pltpu.make_async_copy(
            cenT_hbm.at[:, pl.ds(0, _WC)], win_v.at[0], wsem0).wait()
        soff = process(2 * p, win_v.at[0], soff, False)

        @pl.when(2 * p + 2 < nwin)
        def _():
            pltpu.async_copy(
                cenT_hbm.at[:, pl.ds(
                    pl.multiple_of((wlo + 2 * p + 2) * _WC, _WC), _WC)],
                win_v.at[0], wsem0)

        pltpu.make_async_copy(
            cenT_hbm.at[:, pl.ds(0, _WC)], win_v.at[1], wsem1).wait()
        soff = process(2 * p + 1, win_v.at[1], soff, False)

        @pl.when(2 * p + 3 < nwin)
        def _():
            pltpu.async_copy(
                cenT_hbm.at[:, pl.ds(
                    pl.multiple_of((wlo + 2 * p + 3) * _WC, _WC), _WC)],
                win_v.at[1], wsem1)

        return soff

    soff = lax.fori_loop(0, nwin // 2, pair, 0)
    soff = process(nwin, tail_v, soff, True)

    @pl.when(soff > 0)
    def _():
        flush()


def _tc_body(embT_ref, g_ref, acc_ref, e2_ref, g2_ref):
    i = pl.program_id(0)

    @pl.when(i == 0)
    def _():
        acc_ref[...] = jnp.zeros_like(acc_ref)
        e2_ref[...] = jnp.zeros_like(e2_ref)
        g2_ref[...] = jnp.zeros_like(g2_ref)

    e = embT_ref[...]
    g = g_ref[...]
    acc_ref[...] += lax.dot_general(
        e, g, (((1,), (0,)), ((), ())), preferred_element_type=jnp.float32)
    e2_ref[...] += jnp.sum((e * e).reshape(_D, -1, 128), axis=1)
    g2_ref[...] += jnp.sum((g * g).reshape(8, -1, 128), axis=1)


@jax.jit
def kernel(embeddings, labels, centers):
    labels = labels.astype(jnp.int32)
    emb_t = embeddings.T
    cen_t = centers.T
    tail = jnp.pad(centers[_NWIN * _WC:, :], ((0, 128 - _TAILC), (0, 96)))

    g = pl.kernel(
        _sc_body,
        mesh=plsc.VectorSubcoreMesh(core_axis_name="c", subcore_axis_name="s"),
        compiler_params=pltpu.CompilerParams(needs_layout_passes=False),
        out_type=jax.ShapeDtypeStruct((_GROWS, 128), jnp.float32),
        scratch_types=[
            pltpu.VMEM((_LCHUNK,), jnp.int32),
            pltpu.VMEM((_MCAP,), jnp.int32),
            pltpu.VMEM((_MCAP,), jnp.int32),
            pltpu.VMEM((_MCAP,), jnp.int32),
            pltpu.VMEM((_MCAP,), jnp.int32),
            pltpu.VMEM((_PTRN,), jnp.int32),
            pltpu.VMEM((_PTRN,), jnp.int32),
            pltpu.VMEM((_PTRN,), jnp.int32),
            pltpu.VMEM((2, _D, _WC), jnp.float32),
            pltpu.VMEM((128, 128), jnp.float32),
            pltpu.VMEM((_STAG, 128), jnp.float32),
            pltpu.VMEM((_STAG,), jnp.int32),
            pltpu.VMEM((1, _STAG), jnp.int32),
            pltpu.SemaphoreType.DMA,
            pltpu.SemaphoreType.DMA,
            pltpu.SemaphoreType.DMA,
        ],
    )(labels, cen_t, tail)

    nblk = 8
    bs = _B // nblk
    acc, e2, g2 = pl.pallas_call(
        _tc_body,
        grid=(nblk,),
        in_specs=[
            pl.BlockSpec((_D, bs), lambda i: (0, i)),
            pl.BlockSpec((bs, 128), lambda i: (i, 0)),
        ],
        out_specs=[
            pl.BlockSpec((_D, 128), lambda i: (0, 0)),
            pl.BlockSpec((_D, 128), lambda i: (0, 0)),
            pl.BlockSpec((8, 128), lambda i: (0, 0)),
        ],
        out_shape=[
            jax.ShapeDtypeStruct((_D, 128), jnp.float32),
            jax.ShapeDtypeStruct((_D, 128), jnp.float32),
            jax.ShapeDtypeStruct((8, 128), jnp.float32),
        ],
    )(emb_t, g)

    diag = (jnp.arange(128)[None, :] == jnp.arange(_D)[:, None]).astype(
        jnp.float32)
    tr = jnp.sum(acc * diag)
    return (jnp.sum(e2) + jnp.sum(g2) - 2.0 * tr) * (1.0 / (_B * _D))


# speedup vs baseline: 4.6525x; 1.2180x over previous
import jax
import jax.numpy as jnp
from jax import lax
from jax.experimental import pallas as pl
from jax.experimental.pallas import tpu as pltpu
from jax.experimental.pallas import tpu_sc as plsc

_B = 16384
_D = 32
_V = 1000000
_NC = 2
_NS = 16
_NW = _NC * _NS
_L = 16
_WC = 256
_NWIN = 3906
_TAILC = _V - _NWIN * _WC
_NPAIRG = _NWIN // 2
_LCHUNK = 2048
_MCAP = _B + 32
_STAG = 128
_GROWS = _B + _STAG
_PTRN = 144


def _sc_body(lab_hbm, cenT_hbm, tail_hbm, g_hbm,
             labc_v, mlab_v, midx_v, csr_lab_v, csr_idx_v,
             cnt_v, ptr_v, run_v, win_v, tail_v, stag_v, sidx_v, sidx2_v,
             wsem0, wsem1, wsem2, wsem3, ssem):
    wid = lax.axis_index("s") * _NC + lax.axis_index("c")
    wlo = 2 * ((wid * _NPAIRG) // _NW)
    whi = 2 * (((wid + 1) * _NPAIRG) // _NW)
    nwin = whi - wlo
    lanes = lax.iota(jnp.int32, _L)
    zeros = jnp.zeros((_L,), jnp.float32)
    izeros = jnp.zeros((_L,), jnp.int32)
    dvecs = [jnp.full((_L,), d, jnp.int32) for d in range(_D)]
    wsems = (wsem0, wsem1, wsem2, wsem3)

    def fire(w, j):
        pltpu.async_copy(
            cenT_hbm.at[:, pl.ds(pl.multiple_of((wlo + w) * _WC, _WC), _WC)],
            win_v.at[j], wsems[j])

    def wwait(j):
        pltpu.make_async_copy(
            cenT_hbm.at[:, pl.ds(0, _WC)], win_v.at[j], wsems[j]).wait()

    for j in range(4):
        fire(j, j)
    pltpu.sync_copy(tail_hbm, tail_v)

    def zrow(r, c):
        for k in range(2, 8):
            stag_v[r, pl.ds(k * _L, _L)] = zeros
        return c
    lax.fori_loop(0, _STAG, zrow, 0)
    for k in range(8):
        sidx_v[pl.ds(k * _L, _L)] = _B + ((wid * 4 + k * _L + lanes) % _STAG)
    for k in range(_PTRN // _L):
        cnt_v[pl.ds(k * _L, _L)] = izeros

    rcal, _ = plsc.scan_count(izeros)
    rbase = rcal[0]

    iam_tail = (wid == _NW - 1)

    def scan_chunk(c, moff):
        pltpu.sync_copy(lab_hbm.at[pl.ds(c * _LCHUNK, _LCHUNK)], labc_v)

        def scan_vec(v, moff):
            l = labc_v[pl.ds(v * _L, _L)]
            gw = l >> 8
            m = (gw >= wlo) & (gw < whi)
            m = m | ((gw >= _NWIN) & iam_tail)
            ivec = c * _LCHUNK + v * _L + lanes
            plsc.store_compressed(mlab_v.at[pl.ds(moff, _L)], l, mask=m)
            plsc.store_compressed(midx_v.at[pl.ds(moff, _L)], ivec, mask=m)
            pc = plsc.all_reduce_population_count(m)
            return moff + pc[0]

        return lax.fori_loop(0, _LCHUNK // _L, scan_vec, moff)

    mcnt = lax.fori_loop(0, _B // _LCHUNK, scan_chunk, 0)
    nmv = (mcnt + _L - 1) // _L

    def hist_vec(v, c):
        valid = (v * _L + lanes) < mcnt
        l = mlab_v[pl.ds(v * _L, _L)]
        w = jnp.where(valid, jnp.minimum((l >> 8) - wlo, nwin), 0)
        rank, lastm = plsc.scan_count(w, mask=valid)
        plsc.addupdate_scatter(cnt_v, [w], rank - rbase + 1,
                               mask=lastm & valid)
        return c
    lax.fori_loop(0, nmv, hist_vec, 0)

    def prefix(k, carry):
        c = cnt_v[pl.ds(k * _L, _L)]
        s = plsc.cumsum(c)
        excl = s - c + carry
        ptr_v[pl.ds(k * _L, _L)] = excl
        run_v[pl.ds(k * _L, _L)] = excl
        return carry + s[_L - 1]
    lax.fori_loop(0, _PTRN // _L, prefix, 0)

    def csr_vec(v, c):
        valid = (v * _L + lanes) < mcnt
        l = mlab_v[pl.ds(v * _L, _L)]
        ivec = midx_v[pl.ds(v * _L, _L)]
        w = jnp.where(valid, jnp.minimum((l >> 8) - wlo, nwin), 0)
        rank, lastm = plsc.scan_count(w, mask=valid)
        base = plsc.load_gather(run_v, [w], mask=valid)
        pos = jnp.where(valid, base + rank - rbase, 0)
        plsc.store_scatter(csr_lab_v, [pos], l, mask=valid)
        plsc.store_scatter(csr_idx_v, [pos], ivec, mask=valid)
        plsc.addupdate_scatter(run_v, [w], rank - rbase + 1,
                               mask=lastm & valid)
        return c
    lax.fori_loop(0, nmv, csr_vec, 0)

    def flush():
        for k in range(8):
            sidx2_v[0, pl.ds(k * _L, _L)] = sidx_v[pl.ds(k * _L, _L)]
        pltpu.async_copy(stag_v, g_hbm.at[sidx2_v.at[0]], ssem).wait()
        for k in range(8):
            sidx_v[pl.ds(k * _L, _L)] = _B + ((wid * 4 + k * _L + lanes)
                                              % _STAG)

    def process(wl, buf, soff, tail):
        p0 = ptr_v[pl.ds(wl, _L)][0]
        cnt = cnt_v[pl.ds(wl, _L)][0]
        c0 = (wlo + wl) * _WC

        def entry_vec(v, soff):
            rem = cnt - v * _L
            valid = lanes < rem
            l = csr_lab_v[pl.ds(p0 + v * _L, _L)]
            ivec = csr_idx_v[pl.ds(p0 + v * _L, _L)]
            if tail:
                co = jnp.where(valid, l - _NWIN * _WC, 0)
            else:
                co = jnp.where(valid, l - c0, 0)
            pos = soff + plsc.cumsum(valid.astype(jnp.int32)) - 1
            pos = jnp.where(valid, pos, 0)
            for d in range(_D):
                if tail:
                    vals = plsc.load_gather(buf, [co, dvecs[d]], mask=valid)
                else:
                    vals = plsc.load_gather(buf, [dvecs[d], co], mask=valid)
                plsc.store_scatter(stag_v, [pos, dvecs[d]], vals, mask=valid)
            plsc.store_scatter(sidx_v, [pos], ivec, mask=valid)
            soff = soff + jnp.minimum(rem, _L)

            @pl.when(soff >= _STAG - _L)
            def _():
                flush()

            return jnp.where(soff >= _STAG - _L, 0, soff)

        nv = (cnt + _L - 1) // _L
        return lax.fori_loop(0, nv, entry_vec, soff)

    def quad(q, soff):
        for j in range(4):
            w = 4 * q + j
            wwait(j)
            soff = process(w, win_v.at[j], soff, False)

            @pl.when(w + 4 < nwin)
            def _(w=w, j=j):
                fire(w + 4, j)

        return soff

    soff = lax.fori_loop(0, nwin // 4, quad, 0)

    def leftover(soff):
        wwait(0)
        soff = process(nwin - 2, win_v.at[0], soff, False)
        wwait(1)
        return process(nwin - 1, win_v.at[1], soff, False)

    soff = lax.cond(nwin % 4 == 2, leftover, lambda s: s, soff)
    soff = process(nwin, tail_v, soff, True)

    @pl.when(soff > 0)
    def _():
        flush()


def _tc_body(embT_ref, g_ref, acc_ref, e2_ref, g2_ref):
    i = pl.program_id(0)

    @pl.when(i == 0)
    def _():
        acc_ref[...] = jnp.zeros_like(acc_ref)
        e2_ref[...] = jnp.zeros_like(e2_ref)
        g2_ref[...] = jnp.zeros_like(g2_ref)

    e = embT_ref[...]
    g = g_ref[...]
    acc_ref[...] += lax.dot_general(
        e, g, (((1,), (0,)), ((), ())), preferred_element_type=jnp.float32)
    e2_ref[...] += jnp.sum((e * e).reshape(_D, -1, 128), axis=1)
    g2_ref[...] += jnp.sum((g * g).reshape(8, -1, 128), axis=1)


@jax.jit
def kernel(embeddings, labels, centers):
    labels = labels.astype(jnp.int32)
    emb_t = embeddings.T
    cen_t = centers.T
    tail = jnp.pad(centers[_NWIN * _WC:, :], ((0, 0), (0, 96)))

    g = pl.kernel(
        _sc_body,
        mesh=plsc.VectorSubcoreMesh(core_axis_name="c", subcore_axis_name="s"),
        compiler_params=pltpu.CompilerParams(needs_layout_passes=False),
        out_type=jax.ShapeDtypeStruct((_GROWS, 128), jnp.float32),
        scratch_types=[
            pltpu.VMEM((_LCHUNK,), jnp.int32),
            pltpu.VMEM((_MCAP,), jnp.int32),
            pltpu.VMEM((_MCAP,), jnp.int32),
            pltpu.VMEM((_MCAP,), jnp.int32),
            pltpu.VMEM((_MCAP,), jnp.int32),
            pltpu.VMEM((_PTRN,), jnp.int32),
            pltpu.VMEM((_PTRN,), jnp.int32),
            pltpu.VMEM((_PTRN,), jnp.int32),
            pltpu.VMEM((4, _D, _WC), jnp.float32),
            pltpu.VMEM((_TAILC, 128), jnp.float32),
            pltpu.VMEM((_STAG, 128), jnp.float32),
            pltpu.VMEM((_STAG,), jnp.int32),
            pltpu.VMEM((1, _STAG), jnp.int32),
            pltpu.SemaphoreType.DMA,
            pltpu.SemaphoreType.DMA,
            pltpu.SemaphoreType.DMA,
            pltpu.SemaphoreType.DMA,
            pltpu.SemaphoreType.DMA,
        ],
    )(labels, cen_t, tail)

    nblk = 8
    bs = _B // nblk
    acc, e2, g2 = pl.pallas_call(
        _tc_body,
        grid=(nblk,),
        in_specs=[
            pl.BlockSpec((_D, bs), lambda i: (0, i)),
            pl.BlockSpec((bs, 128), lambda i: (i, 0)),
        ],
        out_specs=[
            pl.BlockSpec((_D, 128), lambda i: (0, 0)),
            pl.BlockSpec((_D, 128), lambda i: (0, 0)),
            pl.BlockSpec((8, 128), lambda i: (0, 0)),
        ],
        out_shape=[
            jax.ShapeDtypeStruct((_D, 128), jnp.float32),
            jax.ShapeDtypeStruct((_D, 128), jnp.float32),
            jax.ShapeDtypeStruct((8, 128), jnp.float32),
        ],
    )(emb_t, g)

    diag = (jnp.arange(128)[None, :] == jnp.arange(_D)[:, None]).astype(
        jnp.float32)
    tr = jnp.sum(acc * diag)
    return (jnp.sum(e2) + jnp.sum(g2) - 2.0 * tr) * (1.0 / (_B * _D))


# speedup vs baseline: 4.9100x; 1.0553x over previous
import jax
import jax.numpy as jnp
from jax import lax
from jax.experimental import pallas as pl
from jax.experimental.pallas import tpu as pltpu
from jax.experimental.pallas import tpu_sc as plsc

_B = 16384
_D = 32
_V = 1000000
_NC = 2
_NS = 16
_NW = _NC * _NS
_L = 16
_WC = 256
_NWIN = 3906
_TAILC = _V - _NWIN * _WC
_NPAIRG = _NWIN // 2
_LCHUNK = 2048
_MCAP = _B + 32
_STAG = 128
_GROWS = _B + _STAG
_PTRN = 144


def _sc_body(lab_hbm, cenT_hbm, tail_hbm, g_hbm,
             labc_v, mlab_v, midx_v, csr_lab_v, csr_idx_v,
             cnt_v, ptr_v, run_v, win_v, tail_v, stag_v, sidx_v, sidx2_v,
             wsem0, wsem1, wsem2, wsem3, ssem):
    wid = lax.axis_index("s") * _NC + lax.axis_index("c")
    wlo = 2 * ((wid * _NPAIRG) // _NW)
    whi = 2 * (((wid + 1) * _NPAIRG) // _NW)
    nwin = whi - wlo
    lanes = lax.iota(jnp.int32, _L)
    zeros = jnp.zeros((_L,), jnp.float32)
    izeros = jnp.zeros((_L,), jnp.int32)
    dvecs = [jnp.full((_L,), d, jnp.int32) for d in range(_D)]
    wsems = (wsem0, wsem1, wsem2, wsem3)

    def fire(w, j):
        pltpu.async_copy(
            cenT_hbm.at[:, pl.ds(pl.multiple_of((wlo + w) * _WC, _WC), _WC)],
            win_v.at[j], wsems[j])

    def wwait(j):
        pltpu.make_async_copy(
            cenT_hbm.at[:, pl.ds(0, _WC)], win_v.at[j], wsems[j]).wait()

    for j in range(4):
        fire(j, j)
    pltpu.sync_copy(tail_hbm, tail_v)

    def zrow(r, c):
        for k in range(2, 8):
            stag_v[r, pl.ds(k * _L, _L)] = zeros
        return c
    lax.fori_loop(0, _STAG, zrow, 0)
    for k in range(8):
        sidx_v[pl.ds(k * _L, _L)] = _B + ((wid * 4 + k * _L + lanes) % _STAG)
    for k in range(_PTRN // _L):
        cnt_v[pl.ds(k * _L, _L)] = izeros

    rcal, _ = plsc.scan_count(izeros)
    rbase = rcal[0]

    iam_tail = (wid == _NW - 1)

    def scan_chunk(c, moff):
        pltpu.sync_copy(lab_hbm.at[pl.ds(c * _LCHUNK, _LCHUNK)], labc_v)

        def scan_vec(v, moff):
            l = labc_v[pl.ds(v * _L, _L)]
            gw = l >> 8
            m = (gw >= wlo) & (gw < whi)
            m = m | ((gw >= _NWIN) & iam_tail)
            ivec = c * _LCHUNK + v * _L + lanes
            plsc.store_compressed(mlab_v.at[pl.ds(moff, _L)], l, mask=m)
            plsc.store_compressed(midx_v.at[pl.ds(moff, _L)], ivec, mask=m)
            pc = plsc.all_reduce_population_count(m)
            return moff + pc[0]

        return lax.fori_loop(0, _LCHUNK // _L, scan_vec, moff)

    mcnt = lax.fori_loop(0, _B // _LCHUNK, scan_chunk, 0)
    nmv = (mcnt + _L - 1) // _L

    def hist_vec(v, c):
        valid = (v * _L + lanes) < mcnt
        l = mlab_v[pl.ds(v * _L, _L)]
        w = jnp.where(valid, jnp.minimum((l >> 8) - wlo, nwin), 0)
        rank, lastm = plsc.scan_count(w, mask=valid)
        plsc.addupdate_scatter(cnt_v, [w], rank - rbase + 1,
                               mask=lastm & valid)
        return c
    lax.fori_loop(0, nmv, hist_vec, 0)

    def prefix(k, carry):
        c = cnt_v[pl.ds(k * _L, _L)]
        s = plsc.cumsum(c)
        excl = s - c + carry
        ptr_v[pl.ds(k * _L, _L)] = excl
        run_v[pl.ds(k * _L, _L)] = excl
        return carry + s[_L - 1]
    lax.fori_loop(0, _PTRN // _L, prefix, 0)

    def csr_vec(v, c):
        valid = (v * _L + lanes) < mcnt
        l = mlab_v[pl.ds(v * _L, _L)]
        ivec = midx_v[pl.ds(v * _L, _L)]
        w = jnp.where(valid, jnp.minimum((l >> 8) - wlo, nwin), 0)
        rank, lastm = plsc.scan_count(w, mask=valid)
        base = plsc.load_gather(run_v, [w], mask=valid)
        pos = jnp.where(valid, base + rank - rbase, 0)
        plsc.store_scatter(csr_lab_v, [pos], l, mask=valid)
        plsc.store_scatter(csr_idx_v, [pos], ivec, mask=valid)
        plsc.addupdate_scatter(run_v, [w], rank - rbase + 1,
                               mask=lastm & valid)
        return c
    lax.fori_loop(0, nmv, csr_vec, 0)

    def flush():
        for k in range(8):
            sidx2_v[0, pl.ds(k * _L, _L)] = sidx_v[pl.ds(k * _L, _L)]
        pltpu.async_copy(stag_v, g_hbm.at[sidx2_v.at[0]], ssem).wait()
        for k in range(8):
            sidx_v[pl.ds(k * _L, _L)] = _B + ((wid * 4 + k * _L + lanes)
                                              % _STAG)

    def process(wl, buf, soff, tail):
        p0 = ptr_v[pl.ds(wl, _L)][0]
        cnt = cnt_v[pl.ds(wl, _L)][0]
        c0 = (wlo + wl) * _WC

        def entry_vec(v, soff):
            rem = cnt - v * _L
            valid = lanes < rem
            l = csr_lab_v[pl.ds(p0 + v * _L, _L)]
            ivec = csr_idx_v[pl.ds(p0 + v * _L, _L)]
            if tail:
                co = jnp.where(valid, l - _NWIN * _WC, 0)
            else:
                co = jnp.where(valid, l - c0, 0)
            pos = soff + plsc.cumsum(valid.astype(jnp.int32)) - 1
            pos = jnp.where(valid, pos, 0)
            for d in range(_D):
                if tail:
                    vals = plsc.load_gather(buf, [co, dvecs[d]], mask=valid)
                else:
                    vals = plsc.load_gather(buf, [dvecs[d], co], mask=valid)
                plsc.store_scatter(stag_v, [pos, dvecs[d]], vals, mask=valid)
            plsc.store_scatter(sidx_v, [pos], ivec, mask=valid)
            soff = soff + jnp.minimum(rem, _L)

            @pl.when(soff >= _STAG - _L)
            def _():
                flush()

            return jnp.where(soff >= _STAG - _L, 0, soff)

        nv = (cnt + _L - 1) // _L
        return lax.fori_loop(0, nv, entry_vec, soff)

    def quad(q, soff):
        for j in range(4):
            w = 4 * q + j
            wwait(j)
            soff = process(w, win_v.at[j], soff, False)

            @pl.when(w + 4 < nwin)
            def _(w=w, j=j):
                fire(w + 4, j)

        return soff

    soff = lax.fori_loop(0, nwin // 4, quad, 0)

    def leftover(soff):
        wwait(0)
        soff = process(nwin - 2, win_v.at[0], soff, False)
        wwait(1)
        return process(nwin - 1, win_v.at[1], soff, False)

    soff = lax.cond(nwin % 4 == 2, leftover, lambda s: s, soff)
    soff = process(nwin, tail_v, soff, True)

    @pl.when(soff > 0)
    def _():
        flush()


def _tc_body(embT_ref, g_ref, out_ref, acc_ref, e2_ref, g2_ref):
    i = pl.program_id(0)

    @pl.when(i == 0)
    def _():
        acc_ref[...] = jnp.zeros_like(acc_ref)
        e2_ref[...] = jnp.zeros_like(e2_ref)
        g2_ref[...] = jnp.zeros_like(g2_ref)

    e = embT_ref[...]
    g = g_ref[...]
    acc_ref[...] += lax.dot_general(
        e, g, (((1,), (0,)), ((), ())), preferred_element_type=jnp.float32)
    e2_ref[...] += jnp.sum((e * e).reshape(_D, -1, 128), axis=1)
    g2_ref[...] += jnp.sum((g * g).reshape(8, -1, 128), axis=1)

    @pl.when(i == pl.num_programs(0) - 1)
    def _():
        diag = (lax.broadcasted_iota(jnp.int32, (_D, 128), 1)
                == lax.broadcasted_iota(jnp.int32, (_D, 128), 0))
        tr = jnp.sum(jnp.where(diag, acc_ref[...], 0.0))
        out_ref[0, 0] = (jnp.sum(e2_ref[...]) + jnp.sum(g2_ref[...])
                         - 2.0 * tr) * (1.0 / (_B * _D))


@jax.jit
def kernel(embeddings, labels, centers):
    labels = labels.astype(jnp.int32)
    emb_t = embeddings.T
    cen_t = centers.T
    tail = jnp.pad(centers[_NWIN * _WC:, :], ((0, 0), (0, 96)))

    g = pl.kernel(
        _sc_body,
        mesh=plsc.VectorSubcoreMesh(core_axis_name="c", subcore_axis_name="s"),
        compiler_params=pltpu.CompilerParams(needs_layout_passes=False),
        out_type=jax.ShapeDtypeStruct((_GROWS, 128), jnp.float32),
        scratch_types=[
            pltpu.VMEM((_LCHUNK,), jnp.int32),
            pltpu.VMEM((_MCAP,), jnp.int32),
            pltpu.VMEM((_MCAP,), jnp.int32),
            pltpu.VMEM((_MCAP,), jnp.int32),
            pltpu.VMEM((_MCAP,), jnp.int32),
            pltpu.VMEM((_PTRN,), jnp.int32),
            pltpu.VMEM((_PTRN,), jnp.int32),
            pltpu.VMEM((_PTRN,), jnp.int32),
            pltpu.VMEM((4, _D, _WC), jnp.float32),
            pltpu.VMEM((_TAILC, 128), jnp.float32),
            pltpu.VMEM((_STAG, 128), jnp.float32),
            pltpu.VMEM((_STAG,), jnp.int32),
            pltpu.VMEM((1, _STAG), jnp.int32),
            pltpu.SemaphoreType.DMA,
            pltpu.SemaphoreType.DMA,
            pltpu.SemaphoreType.DMA,
            pltpu.SemaphoreType.DMA,
            pltpu.SemaphoreType.DMA,
        ],
    )(labels, cen_t, tail)

    nblk = 8
    bs = _B // nblk
    loss = pl.pallas_call(
        _tc_body,
        grid=(nblk,),
        in_specs=[
            pl.BlockSpec((_D, bs), lambda i: (0, i)),
            pl.BlockSpec((bs, 128), lambda i: (i, 0)),
        ],
        out_specs=pl.BlockSpec(memory_space=pltpu.SMEM),
        out_shape=jax.ShapeDtypeStruct((1, 1), jnp.float32),
        scratch_shapes=[
            pltpu.VMEM((_D, 128), jnp.float32),
            pltpu.VMEM((_D, 128), jnp.float32),
            pltpu.VMEM((8, 128), jnp.float32),
        ],
    )(emb_t, g)
    return loss[0, 0]


# speedup vs baseline: 5.1308x; 1.0450x over previous
import jax
import jax.numpy as jnp
from jax import lax
from jax.experimental import pallas as pl
from jax.experimental.pallas import tpu as pltpu
from jax.experimental.pallas import tpu_sc as plsc

_B = 16384
_D = 32
_V = 1000000
_NC = 2
_NS = 16
_NW = _NC * _NS
_L = 16
_WC = 256
_NWIN = 3906
_TAILC = _V - _NWIN * _WC
_NPAIRG = _NWIN // 2
_LCHUNK = 2048
_MCAP = _B + 32
_STAG = 128
_GROWS = _B + _STAG
_PTRN = 144


def _sc_body(lab_hbm, cenT_hbm, tail_hbm, g_hbm,
             labc_v, mlab_v, midx_v, csr_lab_v, csr_idx_v,
             cnt_v, ptr_v, run_v, win_v, tail_v, stag_v, sidx_v, sidx2_v,
             wsem0, wsem1, wsem2, wsem3, ssem):
    wid = lax.axis_index("s") * _NC + lax.axis_index("c")
    wlo = 2 * ((wid * _NPAIRG) // _NW)
    whi = 2 * (((wid + 1) * _NPAIRG) // _NW)
    nwin = whi - wlo
    lanes = lax.iota(jnp.int32, _L)
    zeros = jnp.zeros((_L,), jnp.float32)
    izeros = jnp.zeros((_L,), jnp.int32)
    dvecs = [jnp.full((_L,), d, jnp.int32) for d in range(_D)]
    wsems = (wsem0, wsem1, wsem2, wsem3)

    def fire(w, j):
        pltpu.async_copy(
            cenT_hbm.at[:, pl.ds(pl.multiple_of((wlo + w) * _WC, _WC), _WC)],
            win_v.at[j], wsems[j])

    def wwait(j):
        pltpu.make_async_copy(
            cenT_hbm.at[:, pl.ds(0, _WC)], win_v.at[j], wsems[j]).wait()

    for j in range(4):
        fire(j, j)
    pltpu.sync_copy(tail_hbm, tail_v)

    def zrow(r, c):
        for k in range(2, 8):
            stag_v[r, pl.ds(k * _L, _L)] = zeros
        return c
    lax.fori_loop(0, _STAG, zrow, 0)
    for k in range(8):
        sidx_v[pl.ds(k * _L, _L)] = _B + ((wid * 4 + k * _L + lanes) % _STAG)
    for k in range(_PTRN // _L):
        cnt_v[pl.ds(k * _L, _L)] = izeros

    rcal, _ = plsc.scan_count(izeros)
    rbase = rcal[0]

    iam_tail = (wid == _NW - 1)

    nlc = _B // _LCHUNK
    lab_copies = [
        pltpu.async_copy(lab_hbm.at[pl.ds(c, 1)],
                         labc_v.at[pl.ds(c % 2, 1)], ssem)
        for c in range(2)
    ]

    moff = 0
    for c in range(nlc):
        lab_copies[c].wait()

        def scan_vec(v, moff, c=c):
            l = labc_v[c % 2, pl.ds(v * _L, _L)]
            gw = l >> 8
            m = (gw >= wlo) & (gw < whi)
            m = m | ((gw >= _NWIN) & iam_tail)
            ivec = c * _LCHUNK + v * _L + lanes
            plsc.store_compressed(mlab_v.at[pl.ds(moff, _L)], l, mask=m)
            plsc.store_compressed(midx_v.at[pl.ds(moff, _L)], ivec, mask=m)
            pc = plsc.all_reduce_population_count(m)
            return moff + pc[0]

        moff = lax.fori_loop(0, _LCHUNK // _L, scan_vec, moff)
        if c + 2 < nlc:
            lab_copies.append(
                pltpu.async_copy(lab_hbm.at[pl.ds(c + 2, 1)],
                                 labc_v.at[pl.ds(c % 2, 1)], ssem))

    mcnt = moff
    nmv = (mcnt + _L - 1) // _L

    def hist_vec(v, c):
        valid = (v * _L + lanes) < mcnt
        l = mlab_v[pl.ds(v * _L, _L)]
        w = jnp.where(valid, jnp.minimum((l >> 8) - wlo, nwin), 0)
        rank, lastm = plsc.scan_count(w, mask=valid)
        plsc.addupdate_scatter(cnt_v, [w], rank - rbase + 1,
                               mask=lastm & valid)
        return c
    lax.fori_loop(0, nmv, hist_vec, 0)

    def prefix(k, carry):
        c = cnt_v[pl.ds(k * _L, _L)]
        s = plsc.cumsum(c)
        excl = s - c + carry
        ptr_v[pl.ds(k * _L, _L)] = excl
        run_v[pl.ds(k * _L, _L)] = excl
        return carry + s[_L - 1]
    lax.fori_loop(0, _PTRN // _L, prefix, 0)

    def csr_vec(v, c):
        valid = (v * _L + lanes) < mcnt
        l = mlab_v[pl.ds(v * _L, _L)]
        ivec = midx_v[pl.ds(v * _L, _L)]
        w = jnp.where(valid, jnp.minimum((l >> 8) - wlo, nwin), 0)
        rank, lastm = plsc.scan_count(w, mask=valid)
        base = plsc.load_gather(run_v, [w], mask=valid)
        pos = jnp.where(valid, base + rank - rbase, 0)
        plsc.store_scatter(csr_lab_v, [pos], l, mask=valid)
        plsc.store_scatter(csr_idx_v, [pos], ivec, mask=valid)
        plsc.addupdate_scatter(run_v, [w], rank - rbase + 1,
                               mask=lastm & valid)
        return c
    lax.fori_loop(0, nmv, csr_vec, 0)

    def flush():
        for k in range(8):
            sidx2_v[0, pl.ds(k * _L, _L)] = sidx_v[pl.ds(k * _L, _L)]
        pltpu.async_copy(stag_v, g_hbm.at[sidx2_v.at[0]], ssem).wait()
        for k in range(8):
            sidx_v[pl.ds(k * _L, _L)] = _B + ((wid * 4 + k * _L + lanes)
                                              % _STAG)

    def process(wl, buf, soff, tail):
        p0 = ptr_v[pl.ds(wl, _L)][0]
        cnt = cnt_v[pl.ds(wl, _L)][0]
        c0 = (wlo + wl) * _WC

        def entry_vec(v, soff):
            rem = cnt - v * _L
            valid = lanes < rem
            l = csr_lab_v[pl.ds(p0 + v * _L, _L)]
            ivec = csr_idx_v[pl.ds(p0 + v * _L, _L)]
            if tail:
                co = jnp.where(valid, l - _NWIN * _WC, 0)
            else:
                co = jnp.where(valid, l - c0, 0)
            pos = soff + plsc.cumsum(valid.astype(jnp.int32)) - 1
            pos = jnp.where(valid, pos, 0)
            for d in range(_D):
                if tail:
                    vals = plsc.load_gather(buf, [co, dvecs[d]], mask=valid)
                else:
                    vals = plsc.load_gather(buf, [dvecs[d], co], mask=valid)
                plsc.store_scatter(stag_v, [pos, dvecs[d]], vals, mask=valid)
            plsc.store_scatter(sidx_v, [pos], ivec, mask=valid)
            soff = soff + jnp.minimum(rem, _L)

            @pl.when(soff >= _STAG - _L)
            def _():
                flush()

            return jnp.where(soff >= _STAG - _L, 0, soff)

        nv = (cnt + _L - 1) // _L
        return lax.fori_loop(0, nv, entry_vec, soff)

    def quad(q, soff):
        for j in range(4):
            w = 4 * q + j
            wwait(j)
            soff = process(w, win_v.at[j], soff, False)

            @pl.when(w + 4 < nwin)
            def _(w=w, j=j):
                fire(w + 4, j)

        return soff

    soff = lax.fori_loop(0, nwin // 4, quad, 0)

    def leftover(soff):
        wwait(0)
        soff = process(nwin - 2, win_v.at[0], soff, False)
        wwait(1)
        return process(nwin - 1, win_v.at[1], soff, False)

    soff = lax.cond(nwin % 4 == 2, leftover, lambda s: s, soff)
    soff = process(nwin, tail_v, soff, True)

    @pl.when(soff > 0)
    def _():
        flush()


def _tc_body(embT_ref, g_ref, out_ref, acc_ref, e2_ref, g2_ref):
    i = pl.program_id(0)

    @pl.when(i == 0)
    def _():
        acc_ref[...] = jnp.zeros_like(acc_ref)
        e2_ref[...] = jnp.zeros_like(e2_ref)
        g2_ref[...] = jnp.zeros_like(g2_ref)

    e = embT_ref[...]
    g = g_ref[...]
    acc_ref[...] += lax.dot_general(
        e, g, (((1,), (0,)), ((), ())), preferred_element_type=jnp.float32)
    e2_ref[...] += jnp.sum((e * e).reshape(_D, -1, 128), axis=1)
    g2_ref[...] += jnp.sum((g * g).reshape(8, -1, 128), axis=1)

    @pl.when(i == pl.num_programs(0) - 1)
    def _():
        diag = (lax.broadcasted_iota(jnp.int32, (_D, 128), 1)
                == lax.broadcasted_iota(jnp.int32, (_D, 128), 0))
        tr = jnp.sum(jnp.where(diag, acc_ref[...], 0.0))
        out_ref[0, 0] = (jnp.sum(e2_ref[...]) + jnp.sum(g2_ref[...])
                         - 2.0 * tr) * (1.0 / (_B * _D))


@jax.jit
def kernel(embeddings, labels, centers):
    labels = labels.astype(jnp.int32).reshape(_B // _LCHUNK, _LCHUNK)
    emb_t = embeddings.T
    cen_t = centers.T
    tail = jnp.pad(centers[_NWIN * _WC:, :], ((0, 0), (0, 96)))

    g = pl.kernel(
        _sc_body,
        mesh=plsc.VectorSubcoreMesh(core_axis_name="c", subcore_axis_name="s"),
        compiler_params=pltpu.CompilerParams(needs_layout_passes=False),
        out_type=jax.ShapeDtypeStruct((_GROWS, 128), jnp.float32),
        scratch_types=[
            pltpu.VMEM((2, _LCHUNK), jnp.int32),
            pltpu.VMEM((_MCAP,), jnp.int32),
            pltpu.VMEM((_MCAP,), jnp.int32),
            pltpu.VMEM((_MCAP,), jnp.int32),
            pltpu.VMEM((_MCAP,), jnp.int32),
            pltpu.VMEM((_PTRN,), jnp.int32),
            pltpu.VMEM((_PTRN,), jnp.int32),
            pltpu.VMEM((_PTRN,), jnp.int32),
            pltpu.VMEM((4, _D, _WC), jnp.float32),
            pltpu.VMEM((_TAILC, 128), jnp.float32),
            pltpu.VMEM((_STAG, 128), jnp.float32),
            pltpu.VMEM((_STAG,), jnp.int32),
            pltpu.VMEM((1, _STAG), jnp.int32),
            pltpu.SemaphoreType.DMA,
            pltpu.SemaphoreType.DMA,
            pltpu.SemaphoreType.DMA,
            pltpu.SemaphoreType.DMA,
            pltpu.SemaphoreType.DMA,
        ],
    )(labels, cen_t, tail)

    nblk = 8
    bs = _B // nblk
    loss = pl.pallas_call(
        _tc_body,
        grid=(nblk,),
        in_specs=[
            pl.BlockSpec((_D, bs), lambda i: (0, i)),
            pl.BlockSpec((bs, 128), lambda i: (i, 0)),
        ],
        out_specs=pl.BlockSpec(memory_space=pltpu.SMEM),
        out_shape=jax.ShapeDtypeStruct((1, 1), jnp.float32),
        scratch_shapes=[
            pltpu.VMEM((_D, 128), jnp.float32),
            pltpu.VMEM((_D, 128), jnp.float32),
            pltpu.VMEM((8, 128), jnp.float32),
        ],
    )(emb_t, g)
    return loss[0, 0]


# speedup vs baseline: 5.2257x; 1.0185x over previous
import jax
import jax.numpy as jnp
from jax import lax
from jax.experimental import pallas as pl
from jax.experimental.pallas import tpu as pltpu
from jax.experimental.pallas import tpu_sc as plsc

_B = 16384
_D = 32
_V = 1000000
_NC = 2
_NS = 16
_NW = _NC * _NS
_L = 16
_WC = 256
_NWIN = 3906
_TAILC = _V - _NWIN * _WC
_NPAIRG = _NWIN // 2
_LCHUNK = 2048
_MCAP = _B + 32
_STAG = 128
_GROWS = _B + _STAG
_PTRN = 144


def _sc_body(lab_hbm, cenT_hbm, tail_hbm, g_hbm,
             labc_v, mlab_v, midx_v, csr_lab_v, csr_idx_v,
             cnt_v, ptr_v, run_v, win_v, tail_v, stag_v, sidx_v, sidx2_v,
             wsem0, wsem1, wsem2, wsem3, ssem):
    wid = lax.axis_index("s") * _NC + lax.axis_index("c")
    wlo = 2 * ((wid * _NPAIRG) // _NW)
    whi = 2 * (((wid + 1) * _NPAIRG) // _NW)
    nwin = whi - wlo
    lanes = lax.iota(jnp.int32, _L)
    zeros = jnp.zeros((_L,), jnp.float32)
    izeros = jnp.zeros((_L,), jnp.int32)
    dvecs = [jnp.full((_L,), d, jnp.int32) for d in range(_D)]
    wsems = (wsem0, wsem1, wsem2, wsem3)

    def fire(w, j):
        pltpu.async_copy(
            cenT_hbm.at[:, pl.ds(pl.multiple_of((wlo + w) * _WC, _WC), _WC)],
            win_v.at[j], wsems[j])

    def wwait(j):
        pltpu.make_async_copy(
            cenT_hbm.at[:, pl.ds(0, _WC)], win_v.at[j], wsems[j]).wait()

    for j in range(4):
        fire(j, j)
    pltpu.sync_copy(tail_hbm, tail_v)

    def zrow(r, c):
        for k in range(2, 8):
            stag_v[r, pl.ds(k * _L, _L)] = zeros
        return c
    lax.fori_loop(0, _STAG, zrow, 0)
    for k in range(8):
        sidx_v[pl.ds(k * _L, _L)] = _B + ((wid * 4 + k * _L + lanes) % _STAG)
    for k in range(_PTRN // _L):
        cnt_v[pl.ds(k * _L, _L)] = izeros

    rcal, _ = plsc.scan_count(izeros)
    rbase = rcal[0]

    iam_tail = (wid == _NW - 1)

    nlc = _B // _LCHUNK
    lab_copies = [
        pltpu.async_copy(lab_hbm.at[pl.ds(c, 1)],
                         labc_v.at[pl.ds(c % 2, 1)], ssem)
        for c in range(2)
    ]

    moff = 0
    for c in range(nlc):
        lab_copies[c].wait()

        def scan_vec(v, moff, c=c):
            l = labc_v[c % 2, pl.ds(v * _L, _L)]
            gw = l >> 8
            m = (gw >= wlo) & (gw < whi)
            m = m | ((gw >= _NWIN) & iam_tail)
            ivec = c * _LCHUNK + v * _L + lanes
            plsc.store_compressed(mlab_v.at[pl.ds(moff, _L)], l, mask=m)
            plsc.store_compressed(midx_v.at[pl.ds(moff, _L)], ivec, mask=m)
            pc = plsc.all_reduce_population_count(m)
            return moff + pc[0]

        moff = lax.fori_loop(0, _LCHUNK // _L, scan_vec, moff)
        if c + 2 < nlc:
            lab_copies.append(
                pltpu.async_copy(lab_hbm.at[pl.ds(c + 2, 1)],
                                 labc_v.at[pl.ds(c % 2, 1)], ssem))

    mcnt = moff
    nmv = (mcnt + _L - 1) // _L

    def hist_vec(v, c):
        valid = (v * _L + lanes) < mcnt
        l = mlab_v[pl.ds(v * _L, _L)]
        w = jnp.where(valid, jnp.minimum((l >> 8) - wlo, nwin), 0)
        rank, lastm = plsc.scan_count(w, mask=valid)
        plsc.addupdate_scatter(cnt_v, [w], rank - rbase + 1,
                               mask=lastm & valid)
        return c
    lax.fori_loop(0, nmv, hist_vec, 0)

    def prefix(k, carry):
        c = cnt_v[pl.ds(k * _L, _L)]
        s = plsc.cumsum(c)
        excl = s - c + carry
        ptr_v[pl.ds(k * _L, _L)] = excl
        run_v[pl.ds(k * _L, _L)] = excl
        return carry + s[_L - 1]
    lax.fori_loop(0, _PTRN // _L, prefix, 0)

    def csr_vec(v, c):
        valid = (v * _L + lanes) < mcnt
        l = mlab_v[pl.ds(v * _L, _L)]
        ivec = midx_v[pl.ds(v * _L, _L)]
        w = jnp.where(valid, jnp.minimum((l >> 8) - wlo, nwin), 0)
        rank, lastm = plsc.scan_count(w, mask=valid)
        base = plsc.load_gather(run_v, [w], mask=valid)
        pos = jnp.where(valid, base + rank - rbase, 0)
        plsc.store_scatter(csr_lab_v, [pos], l, mask=valid)
        plsc.store_scatter(csr_idx_v, [pos], ivec, mask=valid)
        plsc.addupdate_scatter(run_v, [w], rank - rbase + 1,
                               mask=lastm & valid)
        return c
    lax.fori_loop(0, nmv, csr_vec, 0)

    def flush():
        for k in range(8):
            sidx2_v[0, pl.ds(k * _L, _L)] = sidx_v[pl.ds(k * _L, _L)]
        pltpu.async_copy(stag_v, g_hbm.at[sidx2_v.at[0]], ssem).wait()
        for k in range(8):
            sidx_v[pl.ds(k * _L, _L)] = _B + ((wid * 4 + k * _L + lanes)
                                              % _STAG)

    def process(wl, buf, soff, tail):
        p0 = ptr_v[pl.ds(wl, _L)][0]
        cnt = cnt_v[pl.ds(wl, _L)][0]
        c0 = (wlo + wl) * _WC

        def entry_vec(v, soff):
            rem = cnt - v * _L
            valid = lanes < rem
            l = csr_lab_v[pl.ds(p0 + v * _L, _L)]
            ivec = csr_idx_v[pl.ds(p0 + v * _L, _L)]
            if tail:
                co = jnp.where(valid, l - _NWIN * _WC, 0)
            else:
                co = jnp.where(valid, l - c0, 0)
            pos = soff + plsc.cumsum(valid.astype(jnp.int32)) - 1
            pos = jnp.where(valid, pos, 0)
            for d in range(_D):
                if tail:
                    vals = plsc.load_gather(buf, [co, dvecs[d]], mask=valid)
                else:
                    vals = plsc.load_gather(buf, [dvecs[d], co], mask=valid)
                plsc.store_scatter(stag_v, [pos, dvecs[d]], vals, mask=valid)
            plsc.store_scatter(sidx_v, [pos], ivec, mask=valid)
            soff = soff + jnp.minimum(rem, _L)

            @pl.when(soff >= _STAG - _L)
            def _():
                flush()

            return jnp.where(soff >= _STAG - _L, 0, soff)

        nv = (cnt + _L - 1) // _L
        return lax.fori_loop(0, nv, entry_vec, soff)

    def quad(q, soff):
        for j in range(4):
            w = 4 * q + j
            wwait(j)
            soff = process(w, win_v.at[j], soff, False)

            @pl.when(w + 4 < nwin)
            def _(w=w, j=j):
                fire(w + 4, j)

        return soff

    soff = lax.fori_loop(0, nwin // 4, quad, 0)

    def leftover(soff):
        wwait(0)
        soff = process(nwin - 2, win_v.at[0], soff, False)
        wwait(1)
        return process(nwin - 1, win_v.at[1], soff, False)

    soff = lax.cond(nwin % 4 == 2, leftover, lambda s: s, soff)
    soff = process(nwin, tail_v, soff, True)

    @pl.when(soff > 0)
    def _():
        flush()


def _tc_body(embT_ref, g_ref, out_ref, acc_ref, e2_ref, g2_ref):
    i = pl.program_id(0)

    @pl.when(i == 0)
    def _():
        acc_ref[...] = jnp.zeros_like(acc_ref)
        e2_ref[...] = jnp.zeros_like(e2_ref)
        g2_ref[...] = jnp.zeros_like(g2_ref)

    e = embT_ref[...]
    g = g_ref[...]
    acc_ref[...] += lax.dot_general(
        e, g, (((1,), (0,)), ((), ())), preferred_element_type=jnp.float32)
    e2_ref[...] += jnp.sum((e * e).reshape(_D, -1, 128), axis=1)
    g2_ref[...] += jnp.sum((g * g).reshape(8, -1, 128), axis=1)

    @pl.when(i == pl.num_programs(0) - 1)
    def _():
        diag = (lax.broadcasted_iota(jnp.int32, (_D, 128), 1)
                == lax.broadcasted_iota(jnp.int32, (_D, 128), 0))
        tr = jnp.sum(jnp.where(diag, acc_ref[...], 0.0))
        out_ref[0, 0] = (jnp.sum(e2_ref[...]) + jnp.sum(g2_ref[...])
                         - 2.0 * tr) * (1.0 / (_B * _D))


@jax.jit
def kernel(embeddings, labels, centers):
    labels = labels.astype(jnp.int32).reshape(_B // _LCHUNK, _LCHUNK)
    emb_t = embeddings.T
    cen_t = centers.T
    tail = jnp.pad(centers[_NWIN * _WC:, :], ((0, 0), (0, 96)))

    g = pl.kernel(
        _sc_body,
        mesh=plsc.VectorSubcoreMesh(core_axis_name="c", subcore_axis_name="s"),
        compiler_params=pltpu.CompilerParams(needs_layout_passes=False),
        out_type=jax.ShapeDtypeStruct((_GROWS, 128), jnp.float32),
        scratch_types=[
            pltpu.VMEM((2, _LCHUNK), jnp.int32),
            pltpu.VMEM((_MCAP,), jnp.int32),
            pltpu.VMEM((_MCAP,), jnp.int32),
            pltpu.VMEM((_MCAP,), jnp.int32),
            pltpu.VMEM((_MCAP,), jnp.int32),
            pltpu.VMEM((_PTRN,), jnp.int32),
            pltpu.VMEM((_PTRN,), jnp.int32),
            pltpu.VMEM((_PTRN,), jnp.int32),
            pltpu.VMEM((4, _D, _WC), jnp.float32),
            pltpu.VMEM((_TAILC, 128), jnp.float32),
            pltpu.VMEM((_STAG, 128), jnp.float32),
            pltpu.VMEM((_STAG,), jnp.int32),
            pltpu.VMEM((1, _STAG), jnp.int32),
            pltpu.SemaphoreType.DMA,
            pltpu.SemaphoreType.DMA,
            pltpu.SemaphoreType.DMA,
            pltpu.SemaphoreType.DMA,
            pltpu.SemaphoreType.DMA,
        ],
    )(labels, cen_t, tail)

    nblk = 4
    bs = _B // nblk
    loss = pl.pallas_call(
        _tc_body,
        grid=(nblk,),
        in_specs=[
            pl.BlockSpec((_D, bs), lambda i: (0, i)),
            pl.BlockSpec((bs, 128), lambda i: (i, 0)),
        ],
        out_specs=pl.BlockSpec(memory_space=pltpu.SMEM),
        out_shape=jax.ShapeDtypeStruct((1, 1), jnp.float32),
        scratch_shapes=[
            pltpu.VMEM((_D, 128), jnp.float32),
            pltpu.VMEM((_D, 128), jnp.float32),
            pltpu.VMEM((8, 128), jnp.float32),
        ],
    )(emb_t, g)
    return loss[0, 0]
